# baseline (speedup 1.0000x reference)
import jax
import jax.numpy as jnp
from jax.experimental import pallas as pl
from jax.experimental.pallas import tpu as pltpu

KTOP = 16
DEPTH = 4


def _fused_kernel(s_ref, xr_ref, xc_ref, q_ref, vals_ref, idx_ref):
    xr = xr_ref[0]
    xc = xc_ref[0]
    q = q_ref[0]
    s = s_ref[0]

    dot = jax.lax.dot_general(
        xr, xc, (((1,), (1,)), ((), ())),
        preferred_element_type=jnp.float32,
        precision=jax.lax.Precision.DEFAULT,
    )
    x2r = jnp.sum(xr * xr, axis=1)[:, None]
    x2c = jnp.sum(xc * xc, axis=1)[None, :]
    d2 = jnp.maximum(x2r + x2c - 2.0 * dot, 0.0)

    score = jnp.log(-jnp.log(q)) - d2 * s

    r, n = score.shape
    lanes = 128
    subs = n // lanes
    neg = -jnp.inf

    w = score.reshape(r, subs, lanes)
    sub_iota = jax.lax.broadcasted_iota(jnp.int32, (r, subs, lanes), 1)
    lane_iota = jax.lax.broadcasted_iota(jnp.int32, (r, 1, lanes), 2)
    cvs = []
    cis = []
    for _ in range(DEPTH):
        m = jnp.max(w, axis=1, keepdims=True)
        s_star = jnp.min(
            jnp.where(w == m, sub_iota, subs), axis=1, keepdims=True
        )
        cvs.append(m)
        cis.append(s_star * lanes + lane_iota)
        w = jnp.where(sub_iota == s_star, neg, w)
    cand_v = jnp.concatenate(cvs, axis=1)
    cand_i = jnp.concatenate(cis, axis=1)

    vals = []
    idxs = []
    for _ in range(KTOP):
        m = jnp.max(cand_v, axis=(1, 2), keepdims=True)
        gidx = jnp.min(
            jnp.where(cand_v == m, cand_i, n), axis=(1, 2), keepdims=True
        )
        vals.append(m[:, :, 0])
        idxs.append(gidx[:, :, 0])
        cand_v = jnp.where(cand_i == gidx, neg, cand_v)
    out_v = jnp.concatenate(vals, axis=1)
    out_i = jnp.concatenate(idxs, axis=1)
    vals_ref[0] = out_v
    idx_ref[0] = out_i

    pc = jnp.sum((cand_v == neg).astype(jnp.int32), axis=1)
    hidden = jnp.max(w, axis=1)
    v16 = out_v[:, KTOP - 1][:, None]
    bad = jnp.any((pc >= DEPTH) & (hidden >= v16))

    @pl.when(bad)
    def _fallback():
        iota = jax.lax.broadcasted_iota(jnp.int32, (r, n), 1)
        cur = score
        fvals = []
        fidxs = []
        for _ in range(KTOP):
            fm = jnp.max(cur, axis=1, keepdims=True)
            fi = jnp.min(jnp.where(cur == fm, iota, n), axis=1, keepdims=True)
            fvals.append(fm)
            fidxs.append(fi)
            cur = jnp.where(iota == fi, neg, cur)
        vals_ref[0] = jnp.concatenate(fvals, axis=1)
        idx_ref[0] = jnp.concatenate(fidxs, axis=1)


def _topk(x, s, q, row_block):
    b, n, d = x.shape
    grid = (b, n // row_block)
    vals, idx = pl.pallas_call(
        _fused_kernel,
        grid=grid,
        in_specs=[
            pl.BlockSpec(memory_space=pltpu.SMEM),
            pl.BlockSpec((1, row_block, d), lambda bi, ri: (bi, ri, 0)),
            pl.BlockSpec((1, n, d), lambda bi, ri: (bi, 0, 0)),
            pl.BlockSpec((1, row_block, n), lambda bi, ri: (bi, ri, 0)),
        ],
        out_specs=[
            pl.BlockSpec((1, row_block, KTOP), lambda bi, ri: (bi, ri, 0)),
            pl.BlockSpec((1, row_block, KTOP), lambda bi, ri: (bi, ri, 0)),
        ],
        out_shape=[
            jax.ShapeDtypeStruct((b, n, KTOP), jnp.float32),
            jax.ShapeDtypeStruct((b, n, KTOP), jnp.int32),
        ],
    )(s, x, x, q)
    return vals, idx


def kernel(x, A, temperature, q):
    b, n, d = x.shape
    s = jnp.exp(jnp.clip(temperature, -5.0, 5.0)).reshape(1)
    logprobs, indices = _topk(x, s, q, 256)

    rows = jnp.broadcast_to(
        jnp.arange(n, dtype=indices.dtype)[None, :, None], (b, n, KTOP)
    )
    edges = jnp.stack((indices.reshape(b, -1), rows.reshape(b, -1)), axis=-2)
    offset = (jnp.arange(b, dtype=indices.dtype) * n)[:, None, None]
    edges_hat = jnp.transpose(edges + offset, (1, 0, 2)).reshape(2, -1)
    return (x, edges_hat, logprobs)


# speedup vs baseline: 3.4595x; 3.4595x over previous
import jax
import jax.numpy as jnp
from jax.experimental import pallas as pl
from jax.experimental.pallas import tpu as pltpu

KTOP = 16
DEPTH = 4


def _fused_kernel(s_ref, xr_ref, xc_ref, q_ref, vals_ref, idx_ref):
    xr = xr_ref[0]
    xc = xc_ref[0]
    q = q_ref[0]
    s = s_ref[0]

    dot = jax.lax.dot_general(
        xr, xc, (((1,), (1,)), ((), ())),
        preferred_element_type=jnp.float32,
        precision=jax.lax.Precision.DEFAULT,
    )
    x2r = jnp.sum(xr * xr, axis=1)[:, None]
    x2c = jnp.sum(xc * xc, axis=1)[None, :]
    d2 = jnp.maximum(x2r + x2c - 2.0 * dot, 0.0)

    score = jnp.log(-jnp.log(q)) - d2 * s

    r, n = score.shape
    lanes = 128
    nchunks = n // lanes
    neg = -jnp.inf

    chunks = [score[:, c * lanes:(c + 1) * lanes] for c in range(nchunks)]
    lane_iota = jax.lax.broadcasted_iota(jnp.int32, (r, lanes), 1)
    cand_v = []
    cand_i = []
    for _ in range(DEPTH):
        m = chunks[0]
        for c in range(1, nchunks):
            m = jnp.maximum(m, chunks[c])
        a = jnp.full((r, lanes), nchunks, jnp.int32)
        for c in range(nchunks - 1, -1, -1):
            a = jnp.where(chunks[c] == m, c, a)
        cand_v.append(m)
        cand_i.append(a * lanes + lane_iota)
        for c in range(nchunks):
            chunks[c] = jnp.where(
                (chunks[c] == m) & (a == c), neg, chunks[c]
            )

    head_v = cand_v[0]
    head_i = cand_i[0]
    depth = jnp.zeros((r, lanes), jnp.int32)
    vals = []
    idxs = []
    for _ in range(KTOP):
        m = jnp.max(head_v, axis=1, keepdims=True)
        gidx = jnp.min(
            jnp.where(head_v == m, head_i, n), axis=1, keepdims=True
        )
        vals.append(m)
        idxs.append(gidx)
        sel = head_i == gidx
        depth = depth + sel.astype(jnp.int32)
        nv = jnp.full((r, lanes), neg, jnp.float32)
        ni = jnp.full((r, lanes), n, jnp.int32)
        for t in range(DEPTH - 1, 0, -1):
            take = depth == t
            nv = jnp.where(take, cand_v[t], nv)
            ni = jnp.where(take, cand_i[t], ni)
        head_v = jnp.where(sel, nv, head_v)
        head_i = jnp.where(sel, ni, head_i)
    out_v = jnp.concatenate(vals, axis=1)
    out_i = jnp.concatenate(idxs, axis=1)
    vals_ref[0] = out_v
    idx_ref[0] = out_i

    hidden = chunks[0]
    for c in range(1, nchunks):
        hidden = jnp.maximum(hidden, chunks[c])
    v16 = out_v[:, KTOP - 1][:, None]
    bad = jnp.any((depth >= DEPTH) & (hidden >= v16))

    @pl.when(bad)
    def _fallback():
        iota = jax.lax.broadcasted_iota(jnp.int32, (r, n), 1)
        cur = score
        fvals = []
        fidxs = []
        for _ in range(KTOP):
            fm = jnp.max(cur, axis=1, keepdims=True)
            fi = jnp.min(jnp.where(cur == fm, iota, n), axis=1, keepdims=True)
            fvals.append(fm)
            fidxs.append(fi)
            cur = jnp.where(iota == fi, neg, cur)
        vals_ref[0] = jnp.concatenate(fvals, axis=1)
        idx_ref[0] = jnp.concatenate(fidxs, axis=1)


def _topk(x, s, q, row_block):
    b, n, d = x.shape
    grid = (b, n // row_block)
    vals, idx = pl.pallas_call(
        _fused_kernel,
        grid=grid,
        in_specs=[
            pl.BlockSpec(memory_space=pltpu.SMEM),
            pl.BlockSpec((1, row_block, d), lambda bi, ri: (bi, ri, 0)),
            pl.BlockSpec((1, n, d), lambda bi, ri: (bi, 0, 0)),
            pl.BlockSpec((1, row_block, n), lambda bi, ri: (bi, ri, 0)),
        ],
        out_specs=[
            pl.BlockSpec((1, row_block, KTOP), lambda bi, ri: (bi, ri, 0)),
            pl.BlockSpec((1, row_block, KTOP), lambda bi, ri: (bi, ri, 0)),
        ],
        out_shape=[
            jax.ShapeDtypeStruct((b, n, KTOP), jnp.float32),
            jax.ShapeDtypeStruct((b, n, KTOP), jnp.int32),
        ],
    )(s, x, x, q)
    return vals, idx


def kernel(x, A, temperature, q):
    b, n, d = x.shape
    s = jnp.exp(jnp.clip(temperature, -5.0, 5.0)).reshape(1)
    logprobs, indices = _topk(x, s, q, 256)

    rows = jnp.broadcast_to(
        jnp.arange(n, dtype=indices.dtype)[None, :, None], (b, n, KTOP)
    )
    edges = jnp.stack((indices.reshape(b, -1), rows.reshape(b, -1)), axis=-2)
    offset = (jnp.arange(b, dtype=indices.dtype) * n)[:, None, None]
    edges_hat = jnp.transpose(edges + offset, (1, 0, 2)).reshape(2, -1)
    return (x, edges_hat, logprobs)
